# Initial kernel scaffold; baseline (speedup 1.0000x reference)
import jax
import jax.numpy as jnp
from jax.experimental import pallas as pl


def kernel(widths, table):
    raise NotImplementedError("write your pallas kernel here")


# speedup vs baseline: 5.0891x; 5.0891x over previous
import functools

import jax
import jax.numpy as jnp
from jax import lax
from jax.experimental import pallas as pl
from jax.experimental.pallas import tpu as pltpu
from jax.experimental.pallas import tpu_sc as plsc

D = 32
NW = 32
CHUNK = 128
GROUP = 8


def _make_kernel(n_ch):
    n_groups = n_ch // GROUP
    mesh = plsc.VectorSubcoreMesh(core_axis_name="c", subcore_axis_name="s")

    @functools.partial(
        pl.kernel,
        mesh=mesh,
        out_type=jax.ShapeDtypeStruct((NW, n_ch, CHUNK, D), jnp.float32),
        scratch_types=[
            pltpu.VMEM((GROUP, CHUNK), jnp.int32),
            pltpu.VMEM((GROUP, CHUNK, D), jnp.float32),
            pltpu.SemaphoreType.DMA,
        ],
        compiler_params=pltpu.CompilerParams(use_tc_tiling_on_sc=False),
    )
    def k(idx_hbm, table_hbm, out_hbm, idx_v, rows_v, sem):
        wid = lax.axis_index("s") * 2 + lax.axis_index("c")

        def body(g, carry):
            c0 = g * GROUP
            pltpu.sync_copy(idx_hbm.at[wid, pl.ds(c0, GROUP)], idx_v)
            copies = [
                pltpu.async_copy(table_hbm.at[idx_v.at[j]], rows_v.at[j], sem)
                for j in range(GROUP)
            ]
            for c in copies:
                c.wait()
            pltpu.sync_copy(rows_v, out_hbm.at[wid, pl.ds(c0, GROUP)])
            return carry

        lax.fori_loop(0, n_groups, body, 0)

    return k


def kernel(widths, table):
    B, H = widths.shape
    total = B * H
    n_ch = total // (NW * CHUNK)
    idx = widths.reshape(NW, n_ch, CHUNK)
    out = _make_kernel(n_ch)(idx, table)
    return out.reshape(B, H, D)
